# baseline (speedup 1.0000x reference)
import functools
import math

import numpy as np
import jax
import jax.numpy as jnp
from jax import lax
from jax.experimental import pallas as pl
from jax.experimental.pallas import tpu as pltpu
from jax.experimental.pallas import tpu_sc as plsc

_B, _L, _D = 16, 512, 512
_NW = 32
_PB = _L // _NW
_NV = _D // 16
_GR = 3 * _PB


def _pos_enc(seq_len: int, d: int) -> np.ndarray:
    channels = int(math.ceil(d / 2) * 2)
    inv_freq = 1.0 / (10000 ** (np.arange(0, channels, 2, dtype=np.float32) / channels))
    pos = np.arange(seq_len, dtype=np.float32)
    sin_inp = np.einsum("i,j->ij", pos, inv_freq.astype(np.float32))
    emb = np.stack((np.sin(sin_inp), np.cos(sin_inp)), axis=-1).reshape(seq_len, channels)
    return emb[:, :d].astype(np.float32)


def _fold_body(ws_ref, wl_ref, wp_ref, wproj_ref, b_ref, tab_ref):
    b = b_ref[...]
    tab_ref[0:16, :] = jnp.dot(ws_ref[...], wproj_ref[0:_D, :],
                               preferred_element_type=jnp.float32) + b
    tab_ref[16:48, :] = jnp.dot(wl_ref[...], wproj_ref[_D:2 * _D, :],
                                preferred_element_type=jnp.float32)
    tab_ref[48:112, :] = jnp.dot(wp_ref[...], wproj_ref[2 * _D:3 * _D, :],
                                 preferred_element_type=jnp.float32)


_fold_tables = pl.pallas_call(
    _fold_body,
    out_shape=jax.ShapeDtypeStruct((112, _D), jnp.float32),
)


def _sc_body(cidx_hbm, tab_hbm, pe_hbm, out_hbm,
             c_idx, pe_b, g0, g1, ob0, ob1,
             sem_g0, sem_g1, sem_o0, sem_o1):
    wid = lax.axis_index("s") * 2 + lax.axis_index("c")
    colbase = wid * _PB
    ibase = wid * (_B * _GR)

    pltpu.sync_copy(cidx_hbm.at[pl.ds(ibase, _B * _GR)], c_idx)
    pltpu.sync_copy(pe_hbm.at[pl.ds(colbase, _PB)], pe_b)

    gsets = ((g0, sem_g0), (g1, sem_g1))
    osets = ((ob0, sem_o0), (ob1, sem_o1))

    _NS = 6

    def g_copies(b, which):
        g, sg = gsets[which]
        step = _GR // _NS
        return [pltpu.make_async_copy(
            tab_hbm.at[c_idx.at[pl.ds(b * _GR + k * step, step)]],
            g.at[pl.ds(k * step, step)], sg) for k in range(_NS)]

    def g_start(b, which):
        for c in g_copies(b, which):
            c.start()

    def g_wait(b, which):
        for c in g_copies(b, which):
            c.wait()

    def o_copy(b, which):
        ob, so = osets[which]
        return pltpu.make_async_copy(ob, out_hbm.at[b, pl.ds(colbase, _PB)],
                                     so)

    def compute(which):
        g, _ = gsets[which]
        ob, _ = osets[which]

        @plsc.parallel_loop(0, _PB, step=1, unroll=2)
        def _tok(j):
            r = 3 * j
            for c32 in range(_NV):
                sl = pl.ds(c32 * 16, 16)
                ob[j, sl] = (g[r, sl] + g[r + 1, sl] + g[r + 2, sl]
                             + pe_b[j, sl])

    def chunk(i, b, which):
        g_wait(b, which)

        @pl.when(i >= 1)
        def _drain():
            o_copy(b - 2, which).wait()

        compute(which)

        @pl.when(i < _B // 2 - 1)
        def _prefetch():
            g_start(b + 2, which)

        o_copy(b, which).start()

    g_start(0, 0)
    g_start(1, 1)

    def pair(i, c):
        chunk(i, 2 * i, 0)
        chunk(i, 2 * i + 1, 1)
        return c

    lax.fori_loop(0, _B // 2, pair, 0)
    o_copy(_B - 2, 0).wait()
    o_copy(_B - 1, 1).wait()


_sc_gather = functools.partial(
    pl.kernel,
    out_type=jax.ShapeDtypeStruct((_B, _L, _D), jnp.float32),
    mesh=plsc.VectorSubcoreMesh(core_axis_name="c", subcore_axis_name="s"),
    scratch_types=[
        pltpu.VMEM((_B * _GR,), jnp.int32),
        pltpu.VMEM((_PB, _D), jnp.float32),
        pltpu.VMEM((_GR, _D), jnp.float32),
        pltpu.VMEM((_GR, _D), jnp.float32),
        pltpu.VMEM((_PB, _D), jnp.float32),
        pltpu.VMEM((_PB, _D), jnp.float32),
        pltpu.SemaphoreType.DMA,
        pltpu.SemaphoreType.DMA,
        pltpu.SemaphoreType.DMA,
        pltpu.SemaphoreType.DMA,
    ],
)(_sc_body)

_PE = _pos_enc(_L, _D)


@jax.jit
def _run(strength, length, phrase, Ws, Wl, Wp, W_proj, b_proj):
    s = strength.astype(jnp.int32)
    l = length.astype(jnp.int32)
    p = phrase.astype(jnp.int32)
    cidx = jnp.stack([s, l + 16, p + 48], axis=-1)
    cidx = (cidx.reshape(_B, _NW, _PB, 3).transpose(1, 0, 2, 3)
            .reshape(_NW * _B * _GR))
    tab = _fold_tables(Ws, Wl, Wp, W_proj, b_proj.reshape(1, _D))
    pe = jnp.asarray(_PE)
    return _sc_gather(cidx, tab, pe)


def kernel(strength, length, phrase, Ws, Wl, Wp, W_proj, b_proj):
    return _run(strength, length, phrase, Ws, Wl, Wp, W_proj, b_proj)


# speedup vs baseline: 1.3116x; 1.3116x over previous
import functools
import math

import numpy as np
import jax
import jax.numpy as jnp
from jax import lax
from jax.experimental import pallas as pl
from jax.experimental.pallas import tpu as pltpu
from jax.experimental.pallas import tpu_sc as plsc

_B, _L, _D = 16, 512, 512
_NW = 32
_PB = _L // _NW
_NV = _D // 16
_GR = 2 * _PB


def _pos_enc(seq_len: int, d: int) -> np.ndarray:
    channels = int(math.ceil(d / 2) * 2)
    inv_freq = 1.0 / (10000 ** (np.arange(0, channels, 2, dtype=np.float32) / channels))
    pos = np.arange(seq_len, dtype=np.float32)
    sin_inp = np.einsum("i,j->ij", pos, inv_freq.astype(np.float32))
    emb = np.stack((np.sin(sin_inp), np.cos(sin_inp)), axis=-1).reshape(seq_len, channels)
    return emb[:, :d].astype(np.float32)


def _fold_body(ws_ref, wl_ref, wp_ref, wproj_ref, b_ref, tab_ref):
    b = b_ref[...]
    ps = jnp.dot(ws_ref[...], wproj_ref[0:_D, :],
                 preferred_element_type=jnp.float32) + b
    pl_e = jnp.dot(wl_ref[...], wproj_ref[_D:2 * _D, :],
                   preferred_element_type=jnp.float32)
    tab_ref[0:512, :] = (ps[:, None, :] + pl_e[None, :, :]).reshape(512, _D)
    tab_ref[512:576, :] = jnp.dot(wp_ref[...], wproj_ref[2 * _D:3 * _D, :],
                                  preferred_element_type=jnp.float32)


_fold_tables = pl.pallas_call(
    _fold_body,
    out_shape=jax.ShapeDtypeStruct((576, _D), jnp.float32),
)


def _sc_body(cidx_hbm, tab_hbm, pe_hbm, out_hbm,
             c_idx, pe_b, g0, g1, ob0, ob1,
             sem_g0, sem_g1, sem_o0, sem_o1):
    wid = lax.axis_index("s") * 2 + lax.axis_index("c")
    colbase = wid * _PB
    ibase = wid * (_B * _GR)

    pltpu.sync_copy(cidx_hbm.at[pl.ds(ibase, _B * _GR)], c_idx)
    pltpu.sync_copy(pe_hbm.at[pl.ds(colbase, _PB)], pe_b)

    gsets = ((g0, sem_g0), (g1, sem_g1))
    osets = ((ob0, sem_o0), (ob1, sem_o1))

    _NS = 4

    def g_copies(b, which):
        g, sg = gsets[which]
        step = _GR // _NS
        return [pltpu.make_async_copy(
            tab_hbm.at[c_idx.at[pl.ds(b * _GR + k * step, step)]],
            g.at[pl.ds(k * step, step)], sg) for k in range(_NS)]

    def g_start(b, which):
        for c in g_copies(b, which):
            c.start()

    def g_wait(b, which):
        for c in g_copies(b, which):
            c.wait()

    def o_copy(b, which):
        ob, so = osets[which]
        return pltpu.make_async_copy(ob, out_hbm.at[b, pl.ds(colbase, _PB)],
                                     so)

    def compute(which):
        g, _ = gsets[which]
        ob, _ = osets[which]

        @plsc.parallel_loop(0, _PB, step=1, unroll=2)
        def _tok(j):
            r = 2 * j
            for c32 in range(_NV):
                sl = pl.ds(c32 * 16, 16)
                ob[j, sl] = g[r, sl] + g[r + 1, sl] + pe_b[j, sl]

    def chunk(i, b, which):
        g_wait(b, which)

        @pl.when(i >= 1)
        def _drain():
            o_copy(b - 2, which).wait()

        compute(which)

        @pl.when(i < _B // 2 - 1)
        def _prefetch():
            g_start(b + 2, which)

        o_copy(b, which).start()

    g_start(0, 0)
    g_start(1, 1)

    def pair(i, c):
        chunk(i, 2 * i, 0)
        chunk(i, 2 * i + 1, 1)
        return c

    lax.fori_loop(0, _B // 2, pair, 0)
    o_copy(_B - 2, 0).wait()
    o_copy(_B - 1, 1).wait()


_sc_gather = functools.partial(
    pl.kernel,
    out_type=jax.ShapeDtypeStruct((_B, _L, _D), jnp.float32),
    mesh=plsc.VectorSubcoreMesh(core_axis_name="c", subcore_axis_name="s"),
    scratch_types=[
        pltpu.VMEM((_B * _GR,), jnp.int32),
        pltpu.VMEM((_PB, _D), jnp.float32),
        pltpu.VMEM((_GR, _D), jnp.float32),
        pltpu.VMEM((_GR, _D), jnp.float32),
        pltpu.VMEM((_PB, _D), jnp.float32),
        pltpu.VMEM((_PB, _D), jnp.float32),
        pltpu.SemaphoreType.DMA,
        pltpu.SemaphoreType.DMA,
        pltpu.SemaphoreType.DMA,
        pltpu.SemaphoreType.DMA,
    ],
)(_sc_body)

_PE = _pos_enc(_L, _D)


@jax.jit
def _run(strength, length, phrase, Ws, Wl, Wp, W_proj, b_proj):
    s = strength.astype(jnp.int32)
    l = length.astype(jnp.int32)
    p = phrase.astype(jnp.int32)
    cidx = jnp.stack([s * 32 + l, p + 512], axis=-1)
    cidx = (cidx.reshape(_B, _NW, _PB, 2).transpose(1, 0, 2, 3)
            .reshape(_NW * _B * _GR))
    tab = _fold_tables(Ws, Wl, Wp, W_proj, b_proj.reshape(1, _D))
    pe = jnp.asarray(_PE)
    return _sc_gather(cidx, tab, pe)


def kernel(strength, length, phrase, Ws, Wl, Wp, W_proj, b_proj):
    return _run(strength, length, phrase, Ws, Wl, Wp, W_proj, b_proj)


# speedup vs baseline: 1.3835x; 1.0548x over previous
import functools
import math

import numpy as np
import jax
import jax.numpy as jnp
from jax import lax
from jax.experimental import pallas as pl
from jax.experimental.pallas import tpu as pltpu
from jax.experimental.pallas import tpu_sc as plsc

_B, _L, _D = 16, 512, 512
_NW = 32
_PB = _L // _NW
_NV = _D // 16
_GR = 2 * _PB


def _pos_enc(seq_len: int, d: int) -> np.ndarray:
    channels = int(math.ceil(d / 2) * 2)
    inv_freq = 1.0 / (10000 ** (np.arange(0, channels, 2, dtype=np.float32) / channels))
    pos = np.arange(seq_len, dtype=np.float32)
    sin_inp = np.einsum("i,j->ij", pos, inv_freq.astype(np.float32))
    emb = np.stack((np.sin(sin_inp), np.cos(sin_inp)), axis=-1).reshape(seq_len, channels)
    return emb[:, :d].astype(np.float32)


def _fold_body(ws_ref, wl_ref, wp_ref, wproj_ref, b_ref, tab_ref):
    b = b_ref[...]
    ps = jnp.dot(ws_ref[...], wproj_ref[0:_D, :],
                 preferred_element_type=jnp.float32) + b
    pl_e = jnp.dot(wl_ref[...], wproj_ref[_D:2 * _D, :],
                   preferred_element_type=jnp.float32)
    tab_ref[0:512, :] = (ps[:, None, :] + pl_e[None, :, :]).reshape(512, _D)
    tab_ref[512:576, :] = jnp.dot(wp_ref[...], wproj_ref[2 * _D:3 * _D, :],
                                  preferred_element_type=jnp.float32)


_fold_tables = pl.pallas_call(
    _fold_body,
    out_shape=jax.ShapeDtypeStruct((576, _D), jnp.float32),
)


def _sc_body(cidx_hbm, tab_hbm, pe_hbm, out_hbm,
             c_idx, pe_b, g0, g1, ob0, ob1,
             sem_g0, sem_g1, sem_o0, sem_o1):
    wid = lax.axis_index("s") * 2 + lax.axis_index("c")
    colbase = wid * _PB
    ibase = wid * (_B * _GR)

    pltpu.sync_copy(cidx_hbm.at[pl.ds(ibase, _B * _GR)], c_idx)
    pltpu.sync_copy(pe_hbm.at[pl.ds(colbase, _PB)], pe_b)

    gsets = ((g0, sem_g0), (g1, sem_g1))
    osets = ((ob0, sem_o0), (ob1, sem_o1))

    _NS = 4

    def g_copies(b, which):
        g, sg = gsets[which]
        step = _GR // _NS
        return [pltpu.make_async_copy(
            tab_hbm.at[c_idx.at[pl.ds(b * _GR + k * step, step)]],
            g.at[pl.ds(k * step, step)], sg) for k in range(_NS)]

    def g_start(b, which):
        for c in g_copies(b, which):
            c.start()

    def g_wait(b, which):
        for c in g_copies(b, which):
            c.wait()

    def o_copy(b, which):
        ob, so = osets[which]
        return pltpu.make_async_copy(ob, out_hbm.at[b, pl.ds(colbase, _PB)],
                                     so)

    def compute(which):
        g, _ = gsets[which]
        ob, _ = osets[which]

        mask = jnp.int32(-65536)

        @plsc.parallel_loop(0, _PB, step=1, unroll=2)
        def _tok(j):
            r = 2 * j
            for blk in range(_D // 32):
                sw = pl.ds(blk * 16, 16)
                w1 = g[r, sw]
                w2 = g[r + 1, sw]
                lo = (lax.bitcast_convert_type(lax.shift_left(w1, 16), jnp.float32)
                      + lax.bitcast_convert_type(lax.shift_left(w2, 16), jnp.float32))
                hi = (lax.bitcast_convert_type(jnp.bitwise_and(w1, mask), jnp.float32)
                      + lax.bitcast_convert_type(jnp.bitwise_and(w2, mask), jnp.float32))
                s0 = pl.ds(blk * 32, 16)
                s1 = pl.ds(blk * 32 + 16, 16)
                ob[j, s0] = lo + pe_b[j, s0]
                ob[j, s1] = hi + pe_b[j, s1]

    def chunk(i, b, which):
        g_wait(b, which)

        @pl.when(i >= 1)
        def _drain():
            o_copy(b - 2, which).wait()

        compute(which)

        @pl.when(i < _B // 2 - 1)
        def _prefetch():
            g_start(b + 2, which)

        o_copy(b, which).start()

    g_start(0, 0)
    g_start(1, 1)

    def pair(i, c):
        chunk(i, 2 * i, 0)
        chunk(i, 2 * i + 1, 1)
        return c

    lax.fori_loop(0, _B // 2, pair, 0)
    o_copy(_B - 2, 0).wait()
    o_copy(_B - 1, 1).wait()


_sc_gather = functools.partial(
    pl.kernel,
    out_type=jax.ShapeDtypeStruct((_B, _L, _D), jnp.float32),
    mesh=plsc.VectorSubcoreMesh(core_axis_name="c", subcore_axis_name="s"),
    scratch_types=[
        pltpu.VMEM((_B * _GR,), jnp.int32),
        pltpu.VMEM((_PB, _D), jnp.float32),
        pltpu.VMEM((_GR, _D // 2), jnp.int32),
        pltpu.VMEM((_GR, _D // 2), jnp.int32),
        pltpu.VMEM((_PB, _D), jnp.float32),
        pltpu.VMEM((_PB, _D), jnp.float32),
        pltpu.SemaphoreType.DMA,
        pltpu.SemaphoreType.DMA,
        pltpu.SemaphoreType.DMA,
        pltpu.SemaphoreType.DMA,
    ],
)(_sc_body)

_PE = _pos_enc(_L, _D)


@jax.jit
def _run(strength, length, phrase, Ws, Wl, Wp, W_proj, b_proj):
    s = strength.astype(jnp.int32)
    l = length.astype(jnp.int32)
    p = phrase.astype(jnp.int32)
    cidx = jnp.stack([s * 32 + l, p + 512], axis=-1)
    cidx = (cidx.reshape(_B, _NW, _PB, 2).transpose(1, 0, 2, 3)
            .reshape(_NW * _B * _GR))
    tab = _fold_tables(Ws, Wl, Wp, W_proj, b_proj.reshape(1, _D))
    tab = (tab.reshape(576, 16, 2, 16).transpose(0, 1, 3, 2)
           .reshape(576, _D // 2, 2).astype(jnp.bfloat16))
    tab = lax.bitcast_convert_type(tab, jnp.int32)
    pe = jnp.asarray(_PE)
    return _sc_gather(cidx, tab, pe)


def kernel(strength, length, phrase, Ws, Wl, Wp, W_proj, b_proj):
    return _run(strength, length, phrase, Ws, Wl, Wp, W_proj, b_proj)


# speedup vs baseline: 1.4298x; 1.0334x over previous
import functools
import math

import numpy as np
import jax
import jax.numpy as jnp
from jax import lax
from jax.experimental import pallas as pl
from jax.experimental.pallas import tpu as pltpu
from jax.experimental.pallas import tpu_sc as plsc

_B, _L, _D = 16, 512, 512
_NW = 32
_PB = _L // _NW
_NV = _D // 16
_GR = 2 * _PB


def _pos_enc(seq_len: int, d: int) -> np.ndarray:
    channels = int(math.ceil(d / 2) * 2)
    inv_freq = 1.0 / (10000 ** (np.arange(0, channels, 2, dtype=np.float32) / channels))
    pos = np.arange(seq_len, dtype=np.float32)
    sin_inp = np.einsum("i,j->ij", pos, inv_freq.astype(np.float32))
    emb = np.stack((np.sin(sin_inp), np.cos(sin_inp)), axis=-1).reshape(seq_len, channels)
    return emb[:, :d].astype(np.float32)


def _fold_body(ws_ref, wl_ref, wp_ref, wproj_ref, b_ref, tab_ref):
    b = b_ref[...]
    ps = jnp.dot(ws_ref[...], wproj_ref[0:_D, :],
                 preferred_element_type=jnp.float32) + b
    pl_e = jnp.dot(wl_ref[...], wproj_ref[_D:2 * _D, :],
                   preferred_element_type=jnp.float32)
    tab_ref[0:512, :] = (ps[:, None, :] + pl_e[None, :, :]).reshape(512, _D)
    tab_ref[512:576, :] = jnp.dot(wp_ref[...], wproj_ref[2 * _D:3 * _D, :],
                                  preferred_element_type=jnp.float32)


_fold_tables = pl.pallas_call(
    _fold_body,
    out_shape=jax.ShapeDtypeStruct((576, _D), jnp.float32),
)


def _sc_body(cidx_hbm, tab_hbm, pe_hbm, out_hbm,
             c_idx, pe_b, g0, g1, ob0, ob1,
             sem_g0, sem_g1, sem_o0, sem_o1):
    wid = lax.axis_index("s") * 2 + lax.axis_index("c")
    colbase = wid * _PB
    ibase = wid * (_B * _GR)

    pltpu.sync_copy(cidx_hbm.at[pl.ds(ibase, _B * _GR)], c_idx)
    pltpu.sync_copy(pe_hbm.at[pl.ds(colbase, _PB)], pe_b)

    gsets = ((g0, sem_g0), (g1, sem_g1))
    osets = ((ob0, sem_o0), (ob1, sem_o1))

    _CR = 2 * _GR
    _NS = 4

    def g_copies(b, which):
        g, sg = gsets[which]
        step = _CR // _NS
        return [pltpu.make_async_copy(
            tab_hbm.at[c_idx.at[pl.ds(b * _CR + k * step, step)]],
            g.at[pl.ds(k * step, step)], sg) for k in range(_NS)]

    def g_start(b, which):
        for c in g_copies(b, which):
            c.start()

    def g_wait(b, which):
        for c in g_copies(b, which):
            c.wait()

    def o_copies(b, which):
        ob, so = osets[which]
        return [pltpu.make_async_copy(
            ob.at[pl.ds(h * _PB, _PB)],
            out_hbm.at[2 * b + h, pl.ds(colbase, _PB)], so) for h in (0, 1)]

    def o_start(b, which):
        for c in o_copies(b, which):
            c.start()

    def o_wait(b, which):
        for c in o_copies(b, which):
            c.wait()

    def compute(which):
        g, _ = gsets[which]
        ob, _ = osets[which]

        mask = jnp.int32(-65536)

        @plsc.parallel_loop(0, 2 * _PB, step=1, unroll=2)
        def _tok(j):
            r = 2 * j
            jp = jnp.bitwise_and(j, _PB - 1)
            for blk in range(_D // 32):
                sw = pl.ds(blk * 16, 16)
                w1 = g[r, sw]
                w2 = g[r + 1, sw]
                lo = (lax.bitcast_convert_type(lax.shift_left(w1, 16), jnp.float32)
                      + lax.bitcast_convert_type(lax.shift_left(w2, 16), jnp.float32))
                hi = (lax.bitcast_convert_type(jnp.bitwise_and(w1, mask), jnp.float32)
                      + lax.bitcast_convert_type(jnp.bitwise_and(w2, mask), jnp.float32))
                s0 = pl.ds(blk * 32, 16)
                s1 = pl.ds(blk * 32 + 16, 16)
                ob[j, s0] = lo + pe_b[jp, s0]
                ob[j, s1] = hi + pe_b[jp, s1]

    _NB = _B // 2

    def chunk(i, b, which):
        g_wait(b, which)

        @pl.when(i >= 1)
        def _drain():
            o_wait(b - 2, which)

        compute(which)

        @pl.when(i < _NB // 2 - 1)
        def _prefetch():
            g_start(b + 2, which)

        o_start(b, which)

    g_start(0, 0)
    g_start(1, 1)

    def pair(i, c):
        chunk(i, 2 * i, 0)
        chunk(i, 2 * i + 1, 1)
        return c

    lax.fori_loop(0, _NB // 2, pair, 0)
    o_wait(_NB - 2, 0)
    o_wait(_NB - 1, 1)


_sc_gather = functools.partial(
    pl.kernel,
    out_type=jax.ShapeDtypeStruct((_B, _L, _D), jnp.float32),
    mesh=plsc.VectorSubcoreMesh(core_axis_name="c", subcore_axis_name="s"),
    scratch_types=[
        pltpu.VMEM((_B * _GR,), jnp.int32),
        pltpu.VMEM((_PB, _D), jnp.float32),
        pltpu.VMEM((4 * _PB, _D // 2), jnp.int32),
        pltpu.VMEM((4 * _PB, _D // 2), jnp.int32),
        pltpu.VMEM((2 * _PB, _D), jnp.float32),
        pltpu.VMEM((2 * _PB, _D), jnp.float32),
        pltpu.SemaphoreType.DMA,
        pltpu.SemaphoreType.DMA,
        pltpu.SemaphoreType.DMA,
        pltpu.SemaphoreType.DMA,
    ],
)(_sc_body)

_PE = _pos_enc(_L, _D)


@jax.jit
def _run(strength, length, phrase, Ws, Wl, Wp, W_proj, b_proj):
    s = strength.astype(jnp.int32)
    l = length.astype(jnp.int32)
    p = phrase.astype(jnp.int32)
    cidx = jnp.stack([s * 32 + l, p + 512], axis=-1)
    cidx = (cidx.reshape(_B, _NW, _PB, 2).transpose(1, 0, 2, 3)
            .reshape(_NW * _B * _GR))
    tab = _fold_tables(Ws, Wl, Wp, W_proj, b_proj.reshape(1, _D))
    tab = (tab.reshape(576, 16, 2, 16).transpose(0, 1, 3, 2)
           .reshape(576, _D // 2, 2).astype(jnp.bfloat16))
    tab = lax.bitcast_convert_type(tab, jnp.int32)
    pe = jnp.asarray(_PE)
    return _sc_gather(cidx, tab, pe)


def kernel(strength, length, phrase, Ws, Wl, Wp, W_proj, b_proj):
    return _run(strength, length, phrase, Ws, Wl, Wp, W_proj, b_proj)
